# baseline (speedup 1.0000x reference)
import functools

import jax
import jax.numpy as jnp
from jax import lax
from jax.experimental import pallas as pl
from jax.experimental.pallas import tpu as pltpu
from jax.experimental.pallas import tpu_sc as plsc

_L, _B = 200, 4096
_D = 64
_LP = 208
_HALF = _LP // 2
_NC, _NS = 2, 16
_NW = _NC * _NS
_BPW = _B // _NW


@functools.partial(
    pl.kernel,
    mesh=plsc.VectorSubcoreMesh(core_axis_name="c", subcore_axis_name="s"),
    compiler_params=pltpu.CompilerParams(use_tc_tiling_on_sc=False),
    out_type=jax.ShapeDtypeStruct((_B, _D), jnp.float32),
    scratch_types=[
        pltpu.VMEM((_BPW, 2, _HALF), jnp.int32),
        pltpu.VMEM((_BPW, _LP), jnp.float32),
        pltpu.VMEM((_LP, _D), jnp.float32),
        pltpu.VMEM((_BPW, _D), jnp.float32),
        pltpu.SemaphoreType.DMA,
    ],
)
def _emb_avg(seq_hbm, mask_hbm, table_hbm, out_hbm,
             seq_v, mask_v, rows_v, out_v, sem):
    wid = lax.axis_index("s") * _NC + lax.axis_index("c")
    base = wid * _BPW
    pltpu.sync_copy(seq_hbm.at[pl.ds(base, _BPW)], seq_v)
    pltpu.sync_copy(mask_hbm.at[pl.ds(base, _BPW)], mask_v)

    def b_body(b, carry):
        cp0 = pltpu.async_copy(table_hbm.at[seq_v.at[b, 0]],
                               rows_v.at[pl.ds(0, _HALF)], sem)
        cp1 = pltpu.async_copy(table_hbm.at[seq_v.at[b, 1]],
                               rows_v.at[pl.ds(_HALF, _HALF)], sem)
        cp0.wait()
        cp1.wait()

        def c_body(c, acc):
            a0, a1, a2, a3, ms = acc
            mvec = mask_v[b, pl.ds(c * 16, 16)]
            ms = ms + mvec
            for r in range(16):
                row = c * 16 + r
                m = mvec[r]
                a0 = a0 + rows_v[row, pl.ds(0, 16)] * m
                a1 = a1 + rows_v[row, pl.ds(16, 16)] * m
                a2 = a2 + rows_v[row, pl.ds(32, 16)] * m
                a3 = a3 + rows_v[row, pl.ds(48, 16)] * m
            return a0, a1, a2, a3, ms

        z = jnp.zeros((16,), jnp.float32)
        a0, a1, a2, a3, ms = lax.fori_loop(0, _LP // 16, c_body,
                                           (z, z, z, z, z))
        lane = lax.iota(jnp.int32, 16)
        denom = ms
        for sh in (8, 4, 2, 1):
            denom = denom + denom.at[lane ^ sh].get(
                mode="promise_in_bounds")
        out_v[b, pl.ds(0, 16)] = a0 / denom
        out_v[b, pl.ds(16, 16)] = a1 / denom
        out_v[b, pl.ds(32, 16)] = a2 / denom
        out_v[b, pl.ds(48, 16)] = a3 / denom
        return carry

    lax.fori_loop(0, _BPW, b_body, 0)
    pltpu.sync_copy(out_v, out_hbm.at[pl.ds(base, _BPW)])


def kernel(input_seq, input_mask, table):
    seq_t = jnp.pad(input_seq.T, ((0, 0), (0, _LP - _L)))
    mask_t = jnp.pad(input_mask.T.astype(jnp.float32), ((0, 0), (0, _LP - _L)))
    return _emb_avg(seq_t.reshape(_B, 2, _HALF), mask_t, table)


# speedup vs baseline: 1.5338x; 1.5338x over previous
import functools

import jax
import jax.numpy as jnp
from jax import lax
from jax.experimental import pallas as pl
from jax.experimental.pallas import tpu as pltpu
from jax.experimental.pallas import tpu_sc as plsc

_L, _B = 200, 4096
_D = 64
_LP = 208
_NCHUNK = _LP // 16
_CH = 64
_MAXCH = 4
_CIDX = 272
_NC, _NS = 2, 16
_NW = _NC * _NS
_BPW = _B // _NW


@functools.partial(
    pl.kernel,
    mesh=plsc.VectorSubcoreMesh(core_axis_name="c", subcore_axis_name="s"),
    compiler_params=pltpu.CompilerParams(use_tc_tiling_on_sc=False,
                                         needs_layout_passes=False),
    out_type=jax.ShapeDtypeStruct((_B, _D), jnp.float32),
    scratch_types=[
        pltpu.VMEM((_BPW, _LP), jnp.int32),
        pltpu.VMEM((_CIDX,), jnp.int32),
        pltpu.VMEM((_CIDX,), jnp.int32),
        pltpu.VMEM((_MAXCH, _CH, _D), jnp.float32),
        pltpu.VMEM((_MAXCH, _CH, _D), jnp.float32),
        pltpu.VMEM((_BPW, _D), jnp.float32),
        pltpu.SemaphoreType.DMA,
        pltpu.SemaphoreType.DMA,
    ],
)
def _emb_avg(mseq_hbm, table_hbm, out_hbm,
             mseq_v, cidx0, cidx1, rows0, rows1, out_v, sem0, sem1):
    wid = lax.axis_index("s") * _NC + lax.axis_index("c")
    base = wid * _BPW
    pltpu.sync_copy(mseq_hbm.at[pl.ds(base, _BPW)], mseq_v)

    lane = lax.iota(jnp.int32, 16)

    def compact_and_issue(b, cidx, rows, sem):
        cnt = jnp.int32(0)
        for c in range(_NCHUNK):
            svec = mseq_v[b, pl.ds(c * 16, 16)]
            ok = svec >= 0
            p = jnp.where(ok, jnp.int32(1), jnp.int32(0))
            incl = p
            for sh in (1, 2, 4, 8):
                shifted = incl.at[jnp.maximum(lane - sh, 0)].get(
                    mode="promise_in_bounds")
                incl = incl + jnp.where(lane >= sh, shifted, 0)
            pos = jnp.where(ok, cnt + incl - p, _CIDX - 16 + lane)
            plsc.store_scatter(cidx, [pos], svec)
            cnt = cnt + incl[15]
        first = cidx[pl.ds(0, 16)]
        padvec = jnp.full((16,), 0, jnp.int32) + first[0]
        for j in range(_CH // 16):
            plsc.store_scatter(cidx, [cnt + 16 * j + lane], padvec)
        nch = (cnt + (_CH - 1)) // _CH

        def issue(i, _):
            pltpu.async_copy(table_hbm.at[cidx.at[pl.ds(i * _CH, _CH)]],
                             rows.at[i], sem)
            return 0

        lax.fori_loop(0, nch, issue, 0)
        return cnt, nch

    def drain_accum_store(b, cnt, nch, cidx, rows, sem):
        def drain(i, _):
            pltpu.make_async_copy(table_hbm.at[cidx.at[pl.ds(i * _CH, _CH)]],
                                  rows.at[i], sem).wait()
            return 0

        lax.fori_loop(0, nch, drain, 0)

        def chunk_acc(i, acc):
            a0, a1, a2, a3 = acc
            for r in range(_CH):
                a0 = a0 + rows[i, r, pl.ds(0, 16)]
                a1 = a1 + rows[i, r, pl.ds(16, 16)]
                a2 = a2 + rows[i, r, pl.ds(32, 16)]
                a3 = a3 + rows[i, r, pl.ds(48, 16)]
            return a0, a1, a2, a3

        z = jnp.zeros((16,), jnp.float32)
        a0, a1, a2, a3 = lax.fori_loop(0, nch, chunk_acc, (z, z, z, z))
        padf = (jnp.full((16,), 0, jnp.int32)
                + (nch * _CH - cnt)).astype(jnp.float32)
        cntf = (jnp.full((16,), 0, jnp.int32) + cnt).astype(jnp.float32)
        out_v[b, pl.ds(0, 16)] = (a0 - padf * rows[0, 0, pl.ds(0, 16)]) / cntf
        out_v[b, pl.ds(16, 16)] = (a1 - padf * rows[0, 0, pl.ds(16, 16)]) / cntf
        out_v[b, pl.ds(32, 16)] = (a2 - padf * rows[0, 0, pl.ds(32, 16)]) / cntf
        out_v[b, pl.ds(48, 16)] = (a3 - padf * rows[0, 0, pl.ds(48, 16)]) / cntf

    cnt0, nch0 = compact_and_issue(0, cidx0, rows0, sem0)

    def step(s, carry, issue_next):
        cnt0, nch0 = carry
        b = 2 * s
        cnt1, nch1 = compact_and_issue(b + 1, cidx1, rows1, sem1)
        drain_accum_store(b, cnt0, nch0, cidx0, rows0, sem0)
        if issue_next:
            cnt0, nch0 = compact_and_issue(b + 2, cidx0, rows0, sem0)
        drain_accum_store(b + 1, cnt1, nch1, cidx1, rows1, sem1)
        return cnt0, nch0

    carry = lax.fori_loop(0, _BPW // 2 - 1,
                          functools.partial(step, issue_next=True),
                          (cnt0, nch0))
    step(_BPW // 2 - 1, carry, issue_next=False)

    pltpu.sync_copy(out_v, out_hbm.at[pl.ds(base, _BPW)])


def kernel(input_seq, input_mask, table):
    mseq = jnp.where(input_mask != 0, input_seq, -1).T
    mseq = jnp.pad(mseq, ((0, 0), (0, _LP - _L)), constant_values=-1)
    return _emb_avg(mseq, table)


# speedup vs baseline: 1.6968x; 1.1063x over previous
import functools

import jax
import jax.numpy as jnp
from jax import lax
from jax.experimental import pallas as pl
from jax.experimental.pallas import tpu as pltpu
from jax.experimental.pallas import tpu_sc as plsc

_L, _B = 200, 4096
_VOCAB = 1000000
_D = 64
_LP = 208
_NCHUNK = _LP // 16
_CH = 64
_MAXCH = 4
_CIDX = 272
_NC, _NS = 2, 16
_NW = _NC * _NS
_BPW = _B // _NW


@functools.partial(
    pl.kernel,
    mesh=plsc.VectorSubcoreMesh(core_axis_name="c", subcore_axis_name="s"),
    compiler_params=pltpu.CompilerParams(use_tc_tiling_on_sc=False,
                                         needs_layout_passes=False),
    out_type=jax.ShapeDtypeStruct((_B, _D), jnp.float32),
    scratch_types=[
        pltpu.VMEM((_BPW, _LP), jnp.int32),
        pltpu.VMEM((_CIDX,), jnp.int32),
        pltpu.VMEM((_CIDX,), jnp.int32),
        pltpu.VMEM((_CIDX,), jnp.int32),
        pltpu.VMEM((_CIDX,), jnp.int32),
        pltpu.VMEM((_MAXCH, _CH, _D), jnp.float32),
        pltpu.VMEM((_MAXCH, _CH, _D), jnp.float32),
        pltpu.VMEM((_MAXCH, _CH, _D), jnp.float32),
        pltpu.VMEM((_MAXCH, _CH, _D), jnp.float32),
        pltpu.VMEM((_BPW, _D), jnp.float32),
        pltpu.SemaphoreType.DMA,
        pltpu.SemaphoreType.DMA,
        pltpu.SemaphoreType.DMA,
        pltpu.SemaphoreType.DMA,
    ],
)
def _emb_avg(mseq_hbm, table_hbm, out_hbm,
             mseq_v, cidx0, cidx1, cidx2, cidx3,
             rows0, rows1, rows2, rows3, out_v, sem0, sem1, sem2, sem3):
    wid = lax.axis_index("s") * _NC + lax.axis_index("c")
    base = wid * _BPW
    pltpu.sync_copy(mseq_hbm.at[pl.ds(base, _BPW)], mseq_v)

    lane = lax.iota(jnp.int32, 16)

    def compact_and_issue(b, cidx, rows, sem):
        cnt = jnp.int32(0)
        for c in range(_NCHUNK):
            svec = mseq_v[b, pl.ds(c * 16, 16)]
            ok = svec >= 0
            p = jnp.where(ok, jnp.int32(1), jnp.int32(0))
            incl = p
            for sh in (1, 2, 4, 8):
                shifted = incl.at[jnp.maximum(lane - sh, 0)].get(
                    mode="promise_in_bounds")
                incl = incl + jnp.where(lane >= sh, shifted, 0)
            pos = jnp.where(ok, cnt + incl - p, _CIDX - 16 + lane)
            plsc.store_scatter(cidx, [pos], svec)
            cnt = cnt + incl[15]
        first = cidx[pl.ds(0, 16)]
        padvec = jnp.full((16,), 0, jnp.int32) + first[0]
        for j in range(_CH // 16):
            plsc.store_scatter(cidx, [cnt + 16 * j + lane], padvec)
        nch = (cnt + (_CH - 1)) // _CH

        def issue(i, _):
            pltpu.async_copy(table_hbm.at[cidx.at[pl.ds(i * _CH, _CH)]],
                             rows.at[i], sem)
            return 0

        lax.fori_loop(0, nch, issue, 0)
        return cnt, nch

    def drain_accum_store(b, cnt, nch, cidx, rows, sem):
        def drain(i, _):
            pltpu.make_async_copy(table_hbm.at[cidx.at[pl.ds(i * _CH, _CH)]],
                                  rows.at[i], sem).wait()
            return 0

        lax.fori_loop(0, nch, drain, 0)

        def chunk_acc(i, acc):
            a0, a1, a2, a3 = acc
            for r in range(_CH):
                a0 = a0 + rows[i, r, pl.ds(0, 16)]
                a1 = a1 + rows[i, r, pl.ds(16, 16)]
                a2 = a2 + rows[i, r, pl.ds(32, 16)]
                a3 = a3 + rows[i, r, pl.ds(48, 16)]
            return a0, a1, a2, a3

        z = jnp.zeros((16,), jnp.float32)
        a0, a1, a2, a3 = lax.fori_loop(0, nch, chunk_acc, (z, z, z, z))
        padf = (jnp.full((16,), 0, jnp.int32)
                + (nch * _CH - cnt)).astype(jnp.float32)
        cntf = (jnp.full((16,), 0, jnp.int32) + cnt).astype(jnp.float32)
        out_v[b, pl.ds(0, 16)] = (a0 - padf * rows[0, 0, pl.ds(0, 16)]) / cntf
        out_v[b, pl.ds(16, 16)] = (a1 - padf * rows[0, 0, pl.ds(16, 16)]) / cntf
        out_v[b, pl.ds(32, 16)] = (a2 - padf * rows[0, 0, pl.ds(32, 16)]) / cntf
        out_v[b, pl.ds(48, 16)] = (a3 - padf * rows[0, 0, pl.ds(48, 16)]) / cntf

    bufs = ((cidx0, rows0, sem0), (cidx1, rows1, sem1),
            (cidx2, rows2, sem2), (cidx3, rows3, sem3))

    def ci(b, j):
        return compact_and_issue(b, *bufs[j])

    def das(b, carry_j, j):
        drain_accum_store(b, carry_j[0], carry_j[1], *bufs[j])

    c0 = ci(0, 0)
    c1 = ci(1, 1)

    def step(s, carry, issue_tail):
        c0, c1 = carry
        b = 4 * s
        c2 = ci(b + 2, 2)
        das(b, c0, 0)
        c3 = ci(b + 3, 3)
        das(b + 1, c1, 1)
        if issue_tail:
            c0 = ci(b + 4, 0)
        das(b + 2, c2, 2)
        if issue_tail:
            c1 = ci(b + 5, 1)
        das(b + 3, c3, 3)
        return c0, c1

    carry = lax.fori_loop(0, _BPW // 4 - 1,
                          functools.partial(step, issue_tail=True),
                          (c0, c1))
    step(_BPW // 4 - 1, carry, issue_tail=False)

    pltpu.sync_copy(out_v, out_hbm.at[pl.ds(base, _BPW)])


def kernel(input_seq, input_mask, table):
    mseq = jnp.where(input_mask != 0, input_seq * 2, -1).T
    mseq = jnp.pad(mseq, ((0, 0), (0, _LP - _L)), constant_values=-1)
    tpad = jnp.zeros((_VOCAB, 2 * _D), table.dtype)
    tpad = jax.lax.dynamic_update_slice(tpad, table, (0, 0))
    tpad = tpad.reshape(2 * _VOCAB, _D)
    return _emb_avg(mseq, tpad)
